# baseline (speedup 1.0000x reference)
import functools

import jax
import jax.numpy as jnp
from jax.experimental import pallas as pl
from jax.experimental.pallas import tpu as pltpu

_ALPHA = 0.05
_BETA = 3.0
_K = 2
_EPS = 1e-8


def _encoder_body(x_ref, af_ref, ab_ref, e1_ref, e2_ref, wg1_ref, wg2_ref,
                  wzr_ref, bzr_ref, wc_ref, bc_ref, out_ref, *, B, P, N, C, H):
    f32 = jnp.float32
    bf16 = jnp.bfloat16

    Afb = jnp.maximum(af_ref[...], 0)
    Abb = jnp.maximum(ab_ref[...], 0)
    sf = 1.0 / (jnp.sum(Afb.astype(f32), axis=-1, keepdims=True) + _EPS)
    sb = 1.0 / (jnp.sum(Abb.astype(f32), axis=-1, keepdims=True) + _EPS)

    e1 = e1_ref[...]
    e2 = e2_ref[...]
    wg1 = wg1_ref[...].astype(jnp.bfloat16)
    wg2 = wg2_ref[...].astype(jnp.bfloat16)
    wzr = wzr_ref[...].astype(jnp.bfloat16)
    bzr_v = bzr_ref[...]
    wc = wc_ref[...].astype(jnp.bfloat16)
    bc_v = bc_ref[...]
    ones_col = jnp.ones((N, 1), dtype=bf16)
    D = wg1.shape[0]

    def static_chain(A16, scale, Ym):
        P1 = scale * jnp.dot(A16, Ym.astype(bf16), preferred_element_type=f32)
        H1 = _ALPHA * Ym + (1.0 - _ALPHA) * P1
        P2 = scale * jnp.dot(A16, H1.astype(bf16), preferred_element_type=f32)
        H2 = _ALPHA * Ym + (1.0 - _ALPHA) * P2
        return H1, H2

    def k2(y, p1):
        return _ALPHA * y + (1.0 - _ALPHA) * p1

    def blk(Am, b):
        return Am[:, b * D:(b + 1) * D]

    def step(t, hs):
        xts = [x_ref[t, :, b * C:(b + 1) * C] for b in range(B)]
        inps = [jnp.concatenate([xts[b], hs[b]], axis=-1) for b in range(B)]
        zs, cins = [None] * B, [None] * B
        cd1, cd2 = [None] * B, [None] * B
        sd1, sd2 = [None] * B, [None] * B
        for g in range(0, B, 2):
            grp = range(g, min(g + 2, B))
            Ar_d, Acm_d, rs_d, cs_d, h1d1_d, h1d2_d = {}, {}, {}, {}, {}, {}
            for b in grp:
                inp = inps[b]
                inpb = inp.astype(bf16)
                f1 = jnp.tanh(jnp.dot(inpb, wg1, preferred_element_type=f32)
                              * e1)
                f2 = jnp.tanh(jnp.dot(inpb, wg2, preferred_element_type=f32)
                              * e2)
                f1b = (_BETA * f1).astype(bf16)
                f2b = f2.astype(bf16)
                G1 = jnp.concatenate([f1b, -f2b], axis=-1)
                G2 = jnp.concatenate([f2b, f1b], axis=-1)
                Ma = jax.lax.dot_general(G1, G2, (((1,), (1,)), ((), ())),
                                         preferred_element_type=f32)
                Tb = jnp.tanh(Ma).astype(bf16)
                Ar_d[b] = jnp.maximum(Tb, 0)
                Acm_d[b] = jnp.minimum(Tb, 0)
                y1 = jnp.concatenate([inpb, ones_col], axis=-1)
                V = jnp.dot(Ar_d[b], y1, preferred_element_type=f32)
                U = jnp.dot(Acm_d[b], y1, preferred_element_type=f32)
                rs_d[b] = 1.0 / (V[:, -1:] + _EPS)
                cs_d[b] = 1.0 / (U[:, -1:] - _EPS)
                h1d1_d[b] = k2(inp, rs_d[b] * V[:, :-1])
                h1d2_d[b] = k2(inp, cs_d[b] * U[:, :-1])

            Yg = jnp.concatenate([inps[b] for b in grp], axis=-1)
            H1f, H2f = static_chain(Afb, sf, Yg)
            H1b, H2b = static_chain(Abb, sb, Yg)

            h2d1_d = {b: k2(inps[b], rs_d[b] * jnp.dot(
                Ar_d[b], h1d1_d[b].astype(bf16), preferred_element_type=f32))
                for b in grp}
            h2d2_d = {b: k2(inps[b], cs_d[b] * jnp.dot(
                Acm_d[b], h1d2_d[b].astype(bf16), preferred_element_type=f32))
                for b in grp}

            cbs = {}
            for b in grp:
                lb = b - g
                ho = jnp.concatenate(
                    [inps[b], blk(H1f, lb), blk(H2f, lb), blk(H1b, lb),
                     blk(H2b, lb), h1d1_d[b], h2d1_d[b], h1d2_d[b],
                     h2d2_d[b]], axis=-1).astype(bf16)
                zr = jax.nn.sigmoid(
                    jnp.dot(ho, wzr, preferred_element_type=f32) + bzr_v)
                z = zr[:, :H]
                r = zr[:, H:]
                zs[b] = z
                cin = jnp.concatenate([xts[b], r * hs[b]], axis=-1)
                cins[b] = cin
                cbs[b] = cin.astype(bf16)

            c1d1_d = {b: k2(cins[b], rs_d[b] * jnp.dot(
                Ar_d[b], cbs[b], preferred_element_type=f32)) for b in grp}
            c1d2_d = {b: k2(cins[b], cs_d[b] * jnp.dot(
                Acm_d[b], cbs[b], preferred_element_type=f32)) for b in grp}
            Ycg = jnp.concatenate([cins[b] for b in grp], axis=-1)
            G1f, G2f = static_chain(Afb, sf, Ycg)
            G1b, G2b = static_chain(Abb, sb, Ycg)
            for b in grp:
                lb = b - g
                sd1[b] = (blk(G1f, lb), blk(G2f, lb))
                sd2[b] = (blk(G1b, lb), blk(G2b, lb))
                c2d1 = k2(cins[b], rs_d[b] * jnp.dot(
                    Ar_d[b], c1d1_d[b].astype(bf16),
                    preferred_element_type=f32))
                c2d2 = k2(cins[b], cs_d[b] * jnp.dot(
                    Acm_d[b], c1d2_d[b].astype(bf16),
                    preferred_element_type=f32))
                cd1[b] = (c1d1_d[b], c2d1)
                cd2[b] = (c1d2_d[b], c2d2)

        new_hs = []
        for b in range(B):
            ho = jnp.concatenate(
                [cins[b], sd1[b][0], sd1[b][1], sd2[b][0], sd2[b][1],
                 cd1[b][0], cd1[b][1], cd2[b][0], cd2[b][1]],
                axis=-1).astype(bf16)
            c = jnp.tanh(jnp.dot(ho, wc, preferred_element_type=f32) + bc_v)
            new_hs.append(zs[b] * hs[b] + (1.0 - zs[b]) * c)
        return tuple(new_hs)

    h0 = tuple(jnp.zeros((N, H), dtype=f32) for _ in range(B))
    hf = jax.lax.fori_loop(0, P, step, h0)
    for b in range(B):
        out_ref[b] = hf[b]


def kernel(x, A_fwd, A_bwd, E1, E2, Wg1, Wg2, Wzr, bzr, Wc, bc):
    B, P, N, C = x.shape
    H = Wc.shape[1]
    EMB = E1.shape[1]
    D = C + H
    feat = Wzr.shape[0]

    xr = x.transpose(1, 2, 0, 3).reshape(P, N, B * C)
    af16 = A_fwd.astype(jnp.bfloat16)
    ab16 = A_bwd.astype(jnp.bfloat16)
    bzr2 = bzr.reshape(1, -1)
    bc2 = bc.reshape(1, -1)

    body = functools.partial(_encoder_body, B=B, P=P, N=N, C=C, H=H)
    out = pl.pallas_call(
        body,
        grid=(1,),
        in_specs=[
            pl.BlockSpec((P, N, B * C), lambda i: (0, 0, 0)),
            pl.BlockSpec((N, N), lambda i: (0, 0)),
            pl.BlockSpec((N, N), lambda i: (0, 0)),
            pl.BlockSpec((N, EMB), lambda i: (0, 0)),
            pl.BlockSpec((N, EMB), lambda i: (0, 0)),
            pl.BlockSpec((D, EMB), lambda i: (0, 0)),
            pl.BlockSpec((D, EMB), lambda i: (0, 0)),
            pl.BlockSpec((feat, 2 * H), lambda i: (0, 0)),
            pl.BlockSpec((1, 2 * H), lambda i: (0, 0)),
            pl.BlockSpec((feat, H), lambda i: (0, 0)),
            pl.BlockSpec((1, H), lambda i: (0, 0)),
        ],
        out_specs=pl.BlockSpec((B, N, H), lambda i: (0, 0, 0)),
        out_shape=jax.ShapeDtypeStruct((B, N, H), x.dtype),
    )(xr, af16, ab16, E1, E2, Wg1, Wg2, Wzr, bzr2, Wc, bc2)
    return out


# speedup vs baseline: 1.1579x; 1.1579x over previous
import functools

import jax
import jax.numpy as jnp
from jax.experimental import pallas as pl
from jax.experimental.pallas import tpu as pltpu

_ALPHA = 0.05
_BETA = 3.0
_K = 2
_EPS = 1e-8


def _encoder_body(x_ref, af_ref, ab_ref, e1_ref, e2_ref, wg1_ref, wg2_ref,
                  wzr_ref, bzr_ref, wc_ref, bc_ref, out_ref, *, B, P, N, C, H):
    f32 = jnp.float32
    bf16 = jnp.bfloat16

    Afb = jnp.maximum(af_ref[...], 0)
    Abb = jnp.maximum(ab_ref[...], 0)
    sf = 1.0 / (jnp.sum(Afb.astype(f32), axis=-1, keepdims=True) + _EPS)
    sb = 1.0 / (jnp.sum(Abb.astype(f32), axis=-1, keepdims=True) + _EPS)

    e1 = e1_ref[...]
    e2 = e2_ref[...]
    wg1 = wg1_ref[...].astype(jnp.bfloat16)
    wg2 = wg2_ref[...].astype(jnp.bfloat16)
    wzr = wzr_ref[...].astype(jnp.bfloat16)
    bzr_v = bzr_ref[...]
    wc = wc_ref[...].astype(jnp.bfloat16)
    bc_v = bc_ref[...]
    ones_col = jnp.ones((N, 1), dtype=bf16)
    D = wg1.shape[0]

    def static_chain(A16, scale, Ym):
        P1 = scale * jnp.dot(A16, Ym.astype(bf16), preferred_element_type=f32)
        H1 = _ALPHA * Ym + (1.0 - _ALPHA) * P1
        P2 = scale * jnp.dot(A16, H1.astype(bf16), preferred_element_type=f32)
        H2 = _ALPHA * Ym + (1.0 - _ALPHA) * P2
        return H1, H2

    def k2(y, p1):
        return _ALPHA * y + (1.0 - _ALPHA) * p1

    def blk(Am, b):
        return Am[:, b * D:(b + 1) * D]

    def step(t, hs):
        xts = [x_ref[t, :, b * C:(b + 1) * C] for b in range(B)]
        inps = [jnp.concatenate([xts[b], hs[b]], axis=-1) for b in range(B)]
        zs, cins = [None] * B, [None] * B
        cd1, cd2 = [None] * B, [None] * B
        for g in range(0, B, 2):
            grp = range(g, min(g + 2, B))
            Ar_d, Acm_d, rs_d, cs_d, h1d1_d, h1d2_d = {}, {}, {}, {}, {}, {}
            for b in grp:
                inp = inps[b]
                inpb = inp.astype(bf16)
                f1 = jnp.tanh(jnp.dot(inpb, wg1, preferred_element_type=f32)
                              * e1)
                f2 = jnp.tanh(jnp.dot(inpb, wg2, preferred_element_type=f32)
                              * e2)
                f1b = (_BETA * f1).astype(bf16)
                f2b = f2.astype(bf16)
                G1 = jnp.concatenate([f1b, -f2b], axis=-1)
                G2 = jnp.concatenate([f2b, f1b], axis=-1)
                Ma = jax.lax.dot_general(G1, G2, (((1,), (1,)), ((), ())),
                                         preferred_element_type=f32)
                Tb = jnp.tanh(Ma).astype(bf16)
                Ar_d[b] = jnp.maximum(Tb, 0)
                Acm_d[b] = jnp.minimum(Tb, 0)
                y1 = jnp.concatenate([inpb, ones_col], axis=-1)
                V = jnp.dot(Ar_d[b], y1, preferred_element_type=f32)
                U = jnp.dot(Acm_d[b], y1, preferred_element_type=f32)
                rs_d[b] = 1.0 / (V[:, -1:] + _EPS)
                cs_d[b] = 1.0 / (U[:, -1:] - _EPS)
                h1d1_d[b] = k2(inp, rs_d[b] * V[:, :-1])
                h1d2_d[b] = k2(inp, cs_d[b] * U[:, :-1])

            if g == 0:
                Y0 = jnp.concatenate(inps, axis=-1)
                H1f, H2f = static_chain(Afb, sf, Y0)
                H1b, H2b = static_chain(Abb, sb, Y0)

            h2d1_d = {b: k2(inps[b], rs_d[b] * jnp.dot(
                Ar_d[b], h1d1_d[b].astype(bf16), preferred_element_type=f32))
                for b in grp}
            h2d2_d = {b: k2(inps[b], cs_d[b] * jnp.dot(
                Acm_d[b], h1d2_d[b].astype(bf16), preferred_element_type=f32))
                for b in grp}

            cbs = {}
            for b in grp:
                ho = jnp.concatenate(
                    [inps[b], blk(H1f, b), blk(H2f, b), blk(H1b, b),
                     blk(H2b, b), h1d1_d[b], h2d1_d[b], h1d2_d[b],
                     h2d2_d[b]], axis=-1).astype(bf16)
                zr = jax.nn.sigmoid(
                    jnp.dot(ho, wzr, preferred_element_type=f32) + bzr_v)
                z = zr[:, :H]
                r = zr[:, H:]
                zs[b] = z
                cin = jnp.concatenate([xts[b], r * hs[b]], axis=-1)
                cins[b] = cin
                cbs[b] = cin.astype(bf16)

            c1d1_d = {b: k2(cins[b], rs_d[b] * jnp.dot(
                Ar_d[b], cbs[b], preferred_element_type=f32)) for b in grp}
            c1d2_d = {b: k2(cins[b], cs_d[b] * jnp.dot(
                Acm_d[b], cbs[b], preferred_element_type=f32)) for b in grp}
            for b in grp:
                c2d1 = k2(cins[b], rs_d[b] * jnp.dot(
                    Ar_d[b], c1d1_d[b].astype(bf16),
                    preferred_element_type=f32))
                c2d2 = k2(cins[b], cs_d[b] * jnp.dot(
                    Acm_d[b], c1d2_d[b].astype(bf16),
                    preferred_element_type=f32))
                cd1[b] = (c1d1_d[b], c2d1)
                cd2[b] = (c1d2_d[b], c2d2)

        Yc = jnp.concatenate(cins, axis=-1)
        G1f, G2f = static_chain(Afb, sf, Yc)
        G1b, G2b = static_chain(Abb, sb, Yc)

        new_hs = []
        for b in range(B):
            ho = jnp.concatenate(
                [cins[b], blk(G1f, b), blk(G2f, b), blk(G1b, b), blk(G2b, b),
                 cd1[b][0], cd1[b][1], cd2[b][0], cd2[b][1]],
                axis=-1).astype(bf16)
            c = jnp.tanh(jnp.dot(ho, wc, preferred_element_type=f32) + bc_v)
            new_hs.append(zs[b] * hs[b] + (1.0 - zs[b]) * c)
        return tuple(new_hs)

    h0 = tuple(jnp.zeros((N, H), dtype=f32) for _ in range(B))
    hf = jax.lax.fori_loop(0, P, step, h0)
    for b in range(B):
        out_ref[b] = hf[b]


def kernel(x, A_fwd, A_bwd, E1, E2, Wg1, Wg2, Wzr, bzr, Wc, bc):
    B, P, N, C = x.shape
    H = Wc.shape[1]
    EMB = E1.shape[1]
    D = C + H
    feat = Wzr.shape[0]

    xr = x.transpose(1, 2, 0, 3).reshape(P, N, B * C)
    af16 = A_fwd.astype(jnp.bfloat16)
    ab16 = A_bwd.astype(jnp.bfloat16)
    bzr2 = bzr.reshape(1, -1)
    bc2 = bc.reshape(1, -1)

    body = functools.partial(_encoder_body, B=B, P=P, N=N, C=C, H=H)
    out = pl.pallas_call(
        body,
        grid=(1,),
        in_specs=[
            pl.BlockSpec((P, N, B * C), lambda i: (0, 0, 0)),
            pl.BlockSpec((N, N), lambda i: (0, 0)),
            pl.BlockSpec((N, N), lambda i: (0, 0)),
            pl.BlockSpec((N, EMB), lambda i: (0, 0)),
            pl.BlockSpec((N, EMB), lambda i: (0, 0)),
            pl.BlockSpec((D, EMB), lambda i: (0, 0)),
            pl.BlockSpec((D, EMB), lambda i: (0, 0)),
            pl.BlockSpec((feat, 2 * H), lambda i: (0, 0)),
            pl.BlockSpec((1, 2 * H), lambda i: (0, 0)),
            pl.BlockSpec((feat, H), lambda i: (0, 0)),
            pl.BlockSpec((1, H), lambda i: (0, 0)),
        ],
        out_specs=pl.BlockSpec((B, N, H), lambda i: (0, 0, 0)),
        out_shape=jax.ShapeDtypeStruct((B, N, H), x.dtype),
    )(xr, af16, ab16, E1, E2, Wg1, Wg2, Wzr, bzr2, Wc, bc2)
    return out
